# baseline (speedup 1.0000x reference)
import math

import numpy as np
import jax
import jax.numpy as jnp
from jax.experimental import pallas as pl
from jax.experimental.pallas import tpu as pltpu


def _avg_pool_matrix(size, k):
    p = size // k
    M = np.zeros((p, size), np.float32)
    for i in range(p):
        M[i, i * k:(i + 1) * k] = 1.0 / k
    return M


def _bilinear_matrix(out_size, in_size):
    M = np.zeros((out_size, in_size), np.float32)
    if in_size == 1:
        M[:, 0] = 1.0
        return M
    scale = in_size / out_size
    for h in range(out_size):
        src = max((h + 0.5) * scale - 0.5, 0.0)
        i0 = min(int(math.floor(src)), in_size - 1)
        i1 = min(i0 + 1, in_size - 1)
        frac = src - i0
        M[h, i0] += 1.0 - frac
        M[h, i1] += frac
    return M


def _pyramid_operators(H, W):
    p2s, u2ts, sizes = [], [], []
    for kh, kw in [(H, W), (H // 2, W // 2), (H // 3, W // 3), (H // 6, W // 6)]:
        Ph, Pw = _avg_pool_matrix(H, kh), _avg_pool_matrix(W, kw)
        Uh, Uw = _bilinear_matrix(H, Ph.shape[0]), _bilinear_matrix(W, Pw.shape[0])
        p2s.append(np.kron(Ph, Pw))
        u2ts.append(np.kron(Uh, Uw).T)
        sizes.append(p2s[-1].shape[0])
    P = sum(sizes)
    Ppad = 128 * ((P + 127) // 128)
    p2t = np.zeros((H * W, Ppad), np.float32)
    u2t = np.zeros((Ppad, H * W), np.float32)
    masks = np.zeros((4, 1, Ppad), np.float32)
    off = 0
    for k in range(4):
        p2t[:, off:off + sizes[k]] = p2s[k].T
        u2t[off:off + sizes[k], :] = u2ts[k]
        masks[k, 0, off:off + sizes[k]] = 1.0
        off += sizes[k]
    return p2t, u2t, masks


def _batch_tile(batch, cap=128):
    best = 1
    for bt in range(1, min(batch, cap) + 1):
        if batch % bt == 0 and (batch == 1 or batch // bt >= 2):
            best = bt
    return best


def _spp_body(x_ref, p2t_ref, wstack_ref, wid8_ref, mask_ref, u2t_ref,
              bias_ref, o_ref, f_ref):
    Bt, C, HW = x_ref.shape
    O = wstack_ref.shape[0] // 4

    xf = x_ref[...].reshape(Bt * C, HW)
    g = jnp.dot(xf, p2t_ref[...], preferred_element_type=jnp.float32)

    for b in range(Bt):
        gb = g[b * C:(b + 1) * C, :]
        rb = jnp.dot(wstack_ref[...], gb, preferred_element_type=jnp.float32)
        fb = rb[0:O, :] * mask_ref[0]
        for k in range(1, 4):
            fb = fb + rb[k * O:(k + 1) * O, :] * mask_ref[k]
        f_ref[b * O:(b + 1) * O, :] = fb

    up3 = jnp.dot(f_ref[...], u2t_ref[...],
                  preferred_element_type=jnp.float32).reshape(Bt, O, HW)

    for i in range(Bt // 8):
        xg = xf[i * 8 * C:(i + 1) * 8 * C, :]
        idp = jnp.dot(wid8_ref[...], xg,
                      preferred_element_type=jnp.float32).reshape(8, O, HW)
        o_ref[i * 8:(i + 1) * 8] = (idp + up3[i * 8:(i + 1) * 8]
                                    + bias_ref[...])


def kernel(x, weight, bias):
    B, C, H, W = x.shape
    O = weight.shape[0]
    HW = H * W

    p2t_np, u2t_np, masks_np = _pyramid_operators(H, W)
    Ppad = p2t_np.shape[1]

    w2d = weight.reshape(O, 5 * C).astype(jnp.float32)
    wid8 = jnp.kron(jnp.eye(8, dtype=jnp.float32), w2d[:, :C])
    wstack = jnp.concatenate([w2d[:, (k + 1) * C:(k + 2) * C]
                              for k in range(4)], axis=0)
    bias_col = bias.astype(jnp.float32).reshape(1, O, 1)

    Bt = _batch_tile(B)
    x3 = x.reshape(B, C, HW).astype(jnp.bfloat16)

    out = pl.pallas_call(
        _spp_body,
        out_shape=jax.ShapeDtypeStruct((B, O, HW), jnp.float32),
        grid=(B // Bt,),
        in_specs=[
            pl.BlockSpec((Bt, C, HW), lambda i: (i, 0, 0)),
            pl.BlockSpec((HW, Ppad), lambda i: (0, 0)),
            pl.BlockSpec((4 * O, C), lambda i: (0, 0)),
            pl.BlockSpec((8 * O, 8 * C), lambda i: (0, 0)),
            pl.BlockSpec((4, 1, Ppad), lambda i: (0, 0, 0)),
            pl.BlockSpec((Ppad, HW), lambda i: (0, 0)),
            pl.BlockSpec((1, O, 1), lambda i: (0, 0, 0)),
        ],
        out_specs=pl.BlockSpec((Bt, O, HW), lambda i: (i, 0, 0)),
        scratch_shapes=[
            pltpu.VMEM((Bt * O, Ppad), jnp.float32),
        ],
        compiler_params=pltpu.CompilerParams(
            dimension_semantics=("parallel",)),
    )(x3, jnp.asarray(p2t_np, jnp.bfloat16), wstack,
      wid8.astype(jnp.bfloat16), jnp.asarray(masks_np),
      jnp.asarray(u2t_np), bias_col)

    return out.reshape(B, O, H, W)


# speedup vs baseline: 1.0797x; 1.0797x over previous
import math

import numpy as np
import jax
import jax.numpy as jnp
from jax.experimental import pallas as pl
from jax.experimental.pallas import tpu as pltpu


def _avg_pool_matrix(size, k):
    p = size // k
    M = np.zeros((p, size), np.float32)
    for i in range(p):
        M[i, i * k:(i + 1) * k] = 1.0 / k
    return M


def _bilinear_matrix(out_size, in_size):
    M = np.zeros((out_size, in_size), np.float32)
    if in_size == 1:
        M[:, 0] = 1.0
        return M
    scale = in_size / out_size
    for h in range(out_size):
        src = max((h + 0.5) * scale - 0.5, 0.0)
        i0 = min(int(math.floor(src)), in_size - 1)
        i1 = min(i0 + 1, in_size - 1)
        frac = src - i0
        M[h, i0] += 1.0 - frac
        M[h, i1] += frac
    return M


def _pyramid_operators(H, W):
    p2s, u2ts, sizes = [], [], []
    for kh, kw in [(H, W), (H // 2, W // 2), (H // 3, W // 3), (H // 6, W // 6)]:
        Ph, Pw = _avg_pool_matrix(H, kh), _avg_pool_matrix(W, kw)
        Uh, Uw = _bilinear_matrix(H, Ph.shape[0]), _bilinear_matrix(W, Pw.shape[0])
        p2s.append(np.kron(Ph, Pw))
        u2ts.append(np.kron(Uh, Uw).T)
        sizes.append(p2s[-1].shape[0])
    P = sum(sizes)
    Ppad = 128 * ((P + 127) // 128)
    p2t = np.zeros((H * W, Ppad), np.float32)
    u2t = np.zeros((Ppad, H * W), np.float32)
    masks = np.zeros((4, 1, Ppad), np.float32)
    off = 0
    for k in range(4):
        p2t[:, off:off + sizes[k]] = p2s[k].T
        u2t[off:off + sizes[k], :] = u2ts[k]
        masks[k, 0, off:off + sizes[k]] = 1.0
        off += sizes[k]
    return p2t, u2t, masks


def _batch_tile(batch, cap=128):
    best = 1
    for bt in range(1, min(batch, cap) + 1):
        if batch % bt == 0 and (batch == 1 or batch // bt >= 2):
            best = bt
    return best


def _spp_body(x_ref, p2t_ref, ws4_ref, wid8_ref, mask_ref, u2t_ref,
              bias_ref, o_ref, f_ref):
    Bt, C, HW = x_ref.shape
    O = ws4_ref.shape[0] // 16

    xf = x_ref[...].reshape(Bt * C, HW)
    g = jnp.dot(xf, p2t_ref[...], preferred_element_type=jnp.float32)

    for q in range(Bt // 4):
        gq = g[q * 4 * C:(q + 1) * 4 * C, :]
        rq = jnp.dot(ws4_ref[...], gq, preferred_element_type=jnp.float32)
        for j in range(4):
            b = q * 4 + j
            fb = rq[j * 4 * O:j * 4 * O + O, :] * mask_ref[0]
            for k in range(1, 4):
                fb = fb + (rq[j * 4 * O + k * O:j * 4 * O + (k + 1) * O, :]
                           * mask_ref[k])
            f_ref[b * O:(b + 1) * O, :] = fb

    up3 = jnp.dot(f_ref[...], u2t_ref[...],
                  preferred_element_type=jnp.float32).reshape(Bt, O, HW)

    for i in range(Bt // 8):
        xg = xf[i * 8 * C:(i + 1) * 8 * C, :]
        idp = jnp.dot(wid8_ref[...], xg,
                      preferred_element_type=jnp.float32).reshape(8, O, HW)
        o_ref[i * 8:(i + 1) * 8] = (idp + up3[i * 8:(i + 1) * 8]
                                    + bias_ref[...])


def kernel(x, weight, bias):
    B, C, H, W = x.shape
    O = weight.shape[0]
    HW = H * W

    p2t_np, u2t_np, masks_np = _pyramid_operators(H, W)
    Ppad = p2t_np.shape[1]

    w2d = weight.reshape(O, 5 * C).astype(jnp.float32)
    wid8 = jnp.kron(jnp.eye(8, dtype=jnp.float32), w2d[:, :C])
    wstack = jnp.concatenate([w2d[:, (k + 1) * C:(k + 2) * C]
                              for k in range(4)], axis=0)
    ws4 = jnp.kron(jnp.eye(4, dtype=jnp.float32), wstack)
    bias_col = bias.astype(jnp.float32).reshape(1, O, 1)

    Bt = _batch_tile(B)
    x3 = x.astype(jnp.float32).reshape(B, C, HW)

    out = pl.pallas_call(
        _spp_body,
        out_shape=jax.ShapeDtypeStruct((B, O, HW), jnp.float32),
        grid=(B // Bt,),
        in_specs=[
            pl.BlockSpec((Bt, C, HW), lambda i: (i, 0, 0)),
            pl.BlockSpec((HW, Ppad), lambda i: (0, 0)),
            pl.BlockSpec((16 * O, 4 * C), lambda i: (0, 0)),
            pl.BlockSpec((8 * O, 8 * C), lambda i: (0, 0)),
            pl.BlockSpec((4, 1, Ppad), lambda i: (0, 0, 0)),
            pl.BlockSpec((Ppad, HW), lambda i: (0, 0)),
            pl.BlockSpec((1, O, 1), lambda i: (0, 0, 0)),
        ],
        out_specs=pl.BlockSpec((Bt, O, HW), lambda i: (i, 0, 0)),
        scratch_shapes=[
            pltpu.VMEM((Bt * O, Ppad), jnp.float32),
        ],
        compiler_params=pltpu.CompilerParams(
            dimension_semantics=("parallel",)),
    )(x3, jnp.asarray(p2t_np), ws4, wid8, jnp.asarray(masks_np),
      jnp.asarray(u2t_np), bias_col)

    return out.reshape(B, O, H, W)
